# baseline (speedup 1.0000x reference)
import functools

import jax
import jax.numpy as jnp
from jax import lax
from jax.experimental import pallas as pl
from jax.experimental.pallas import tpu as pltpu
from jax.experimental.pallas import tpu_sc as plsc

B = 16384
F = 26
V = 100000
D = 16
NUM = 13
H1 = 128
H2 = 64
N = B * F


_CHUNK = 128


def _make_sc_gather():
    info = plsc.get_sparse_core_info()
    nc, ns = info.num_cores, info.num_subcores
    nw = nc * ns
    per_w = N // nw
    n_chunks = per_w // _CHUNK
    chunks_per_field = B // _CHUNK
    mesh = plsc.VectorSubcoreMesh(core_axis_name="c", subcore_axis_name="s")

    @functools.partial(
        pl.kernel,
        mesh=mesh,
        out_type=jax.ShapeDtypeStruct((B, F * D), jnp.float32),
        scratch_types=[
            pltpu.VMEM((_CHUNK,), jnp.int32),
            pltpu.VMEM((_CHUNK,), jnp.int32),
            pltpu.VMEM((_CHUNK, D), jnp.float32),
            pltpu.VMEM((_CHUNK, D), jnp.float32),
            pltpu.SemaphoreType.DMA,
            pltpu.SemaphoreType.DMA,
        ],
        compiler_params=pltpu.CompilerParams(use_tc_tiling_on_sc=False),
    )
    def gather_k(table_hbm, idx_hbm, out_hbm, idx0, idx1, rows0, rows1,
                 sem0, sem1):
        wid = lax.axis_index("s") * nc + lax.axis_index("c")
        chunk0 = wid * n_chunks
        idxs = (idx0, idx1)
        rows = (rows0, rows1)
        sems = (sem0, sem1)

        def stage_and_gather(c, b):
            f = c // chunks_per_field
            r0 = (c % chunks_per_field) * _CHUNK
            pltpu.sync_copy(idx_hbm.at[f, pl.ds(r0, _CHUNK)], idxs[b])
            pltpu.async_copy(table_hbm.at[f].at[idxs[b]], rows[b], sems[b])

        def finish(c, b):
            f = c // chunks_per_field
            r0 = (c % chunks_per_field) * _CHUNK
            pltpu.make_async_copy(table_hbm.at[f].at[idxs[b]], rows[b],
                                  sems[b]).wait()
            pltpu.sync_copy(rows[b],
                            out_hbm.at[pl.ds(r0, _CHUNK), pl.ds(f * D, D)])

        stage_and_gather(chunk0, 0)

        def body(jj, carry):
            for b in range(2):
                j = jj * 2 + b

                @pl.when(j + 1 < n_chunks)
                def _():
                    stage_and_gather(chunk0 + j + 1, 1 - b)

                finish(chunk0 + j, b)
            return carry

        lax.fori_loop(0, n_chunks // 2, body, 0)

    return gather_k


_sc_gather = _make_sc_gather()


_BB = 1024


def _mlp_body(emb_ref, xn_ref, w1a_ref, w1b_ref, b1_ref, w2_ref, b2_ref,
              w3_ref, b3_ref, o_ref):
    h = jnp.dot(emb_ref[...], w1a_ref[...], preferred_element_type=jnp.float32)
    h = h + jnp.dot(xn_ref[...], w1b_ref[...], preferred_element_type=jnp.float32)
    h = jnp.maximum(h + b1_ref[...], 0.0)
    h = jnp.maximum(
        jnp.dot(h, w2_ref[...], preferred_element_type=jnp.float32) + b2_ref[...],
        0.0)
    o_ref[...] = (
        jnp.dot(h, w3_ref[...], preferred_element_type=jnp.float32) + b3_ref[...])


def _mlp(emb, x_num, w1a, w1b, b1, w2, b2, w3, b3):
    grid = (B // _BB,)
    full = lambda shape: pl.BlockSpec(shape, lambda i: (0, 0))
    return pl.pallas_call(
        _mlp_body,
        grid=grid,
        in_specs=[
            pl.BlockSpec((_BB, F * D), lambda i: (i, 0)),
            pl.BlockSpec((_BB, NUM), lambda i: (i, 0)),
            full(w1a.shape),
            full(w1b.shape),
            full((1, H1)),
            full(w2.shape),
            full((1, H2)),
            full(w3.shape),
            full((1, 1)),
        ],
        out_specs=pl.BlockSpec((_BB, 1), lambda i: (i, 0)),
        out_shape=jax.ShapeDtypeStruct((B, 1), jnp.float32),
    )(emb, x_num, w1a, w1b, b1.reshape(1, H1), w2, b2.reshape(1, H2), w3,
      b3.reshape(1, 1))


def kernel(x_num, x_cat, emb_tables, W1, b1, W2, b2, W3, b3):
    idx_t = x_cat.astype(jnp.int32).T
    emb = _sc_gather(emb_tables, idx_t)
    w1a = W1[:F * D]
    w1b = W1[F * D:]
    return _mlp(emb, x_num, w1a, w1b, b1, W2, b2, W3, b3)


# speedup vs baseline: 1.0836x; 1.0836x over previous
import functools

import jax
import jax.numpy as jnp
from jax import lax
from jax.experimental import pallas as pl
from jax.experimental.pallas import tpu as pltpu
from jax.experimental.pallas import tpu_sc as plsc

B = 16384
F = 26
V = 100000
D = 16
NUM = 13
H1 = 128
H2 = 64
N = B * F


_VT_FULL = V // 128
_TAIL = V - _VT_FULL * 128
_TP = F * _VT_FULL


def _make_sc_relayout():
    info = plsc.get_sparse_core_info()
    nc, ns = info.num_cores, info.num_subcores
    nw = nc * ns
    per_w = -(-_TP // nw)
    mesh = plsc.VectorSubcoreMesh(core_axis_name="c", subcore_axis_name="s")

    @functools.partial(
        pl.kernel,
        mesh=mesh,
        out_type=jax.ShapeDtypeStruct((F * V * D,), jnp.float32),
        scratch_types=[
            pltpu.VMEM((D, 128), jnp.float32),
            pltpu.VMEM((128 * D,), jnp.float32),
            pltpu.VMEM((_TAIL * D,), jnp.float32),
        ],
        compiler_params=pltpu.CompilerParams(use_tc_tiling_on_sc=True,
                                             needs_layout_passes=False),
    )
    def relayout_k(table_hbm, tail_hbm, out_hbm, tile_v, rows_v, tail_v):
        wid = lax.axis_index("s") * nc + lax.axis_index("c")
        g0 = wid * per_w
        nj = jnp.minimum(per_w, _TP - g0)
        iota = lax.iota(jnp.int32, 16)

        def body(j, carry):
            g = g0 + j
            f = g // _VT_FULL
            v0 = (g % _VT_FULL) * 128
            pltpu.sync_copy(table_hbm.at[f, :, pl.ds(v0, 128)], tile_v)
            for d in range(D):
                for k in range(8):
                    vec = tile_v[d, pl.ds(16 * k, 16)]
                    plsc.store_scatter(rows_v, [iota * D + (256 * k + d)], vec)
            pltpu.sync_copy(rows_v,
                            out_hbm.at[pl.ds(f * (V * D) + v0 * D, 128 * D)])
            return carry

        lax.fori_loop(0, nj, body, 0)

        @pl.when(wid < F)
        def _():
            pltpu.sync_copy(tail_hbm.at[pl.ds(wid * (_TAIL * D), _TAIL * D)],
                            tail_v)
            pltpu.sync_copy(
                tail_v,
                out_hbm.at[pl.ds(wid * (V * D) + _VT_FULL * 128 * D,
                                 _TAIL * D)])

    return relayout_k


_CHUNK = 128


def _make_sc_gather():
    info = plsc.get_sparse_core_info()
    nc, ns = info.num_cores, info.num_subcores
    nw = nc * ns
    per_w = N // nw
    n_chunks = per_w // _CHUNK
    chunks_per_field = B // _CHUNK
    mesh = plsc.VectorSubcoreMesh(core_axis_name="c", subcore_axis_name="s")

    @functools.partial(
        pl.kernel,
        mesh=mesh,
        out_type=jax.ShapeDtypeStruct((B, F * D), jnp.float32),
        scratch_types=[
            pltpu.VMEM((_CHUNK,), jnp.int32),
            pltpu.VMEM((_CHUNK,), jnp.int32),
            pltpu.VMEM((_CHUNK, D), jnp.float32),
            pltpu.VMEM((_CHUNK, D), jnp.float32),
            pltpu.SemaphoreType.DMA,
            pltpu.SemaphoreType.DMA,
        ],
        compiler_params=pltpu.CompilerParams(use_tc_tiling_on_sc=False),
    )
    def gather_k(table_hbm, idx_hbm, out_hbm, idx0, idx1, rows0, rows1,
                 sem0, sem1):
        wid = lax.axis_index("s") * nc + lax.axis_index("c")
        chunk0 = wid * n_chunks
        idxs = (idx0, idx1)
        rows = (rows0, rows1)
        sems = (sem0, sem1)

        def stage_and_gather(c, b):
            f = c // chunks_per_field
            r0 = (c % chunks_per_field) * _CHUNK
            pltpu.sync_copy(idx_hbm.at[f, pl.ds(r0, _CHUNK)], idxs[b])
            pltpu.async_copy(table_hbm.at[f].at[idxs[b]], rows[b], sems[b])

        def finish(c, b):
            f = c // chunks_per_field
            r0 = (c % chunks_per_field) * _CHUNK
            pltpu.make_async_copy(table_hbm.at[f].at[idxs[b]], rows[b],
                                  sems[b]).wait()
            pltpu.sync_copy(rows[b],
                            out_hbm.at[pl.ds(r0, _CHUNK), pl.ds(f * D, D)])

        stage_and_gather(chunk0, 0)

        def body(jj, carry):
            for b in range(2):
                j = jj * 2 + b

                @pl.when(j + 1 < n_chunks)
                def _():
                    stage_and_gather(chunk0 + j + 1, 1 - b)

                finish(chunk0 + j, b)
            return carry

        lax.fori_loop(0, n_chunks // 2, body, 0)

    return gather_k


_sc_gather = _make_sc_gather()


_BB = 1024


def _mlp_body(emb_ref, xn_ref, w1a_ref, w1b_ref, b1_ref, w2_ref, b2_ref,
              w3_ref, b3_ref, o_ref):
    h = jnp.dot(emb_ref[...], w1a_ref[...], preferred_element_type=jnp.float32)
    h = h + jnp.dot(xn_ref[...], w1b_ref[...], preferred_element_type=jnp.float32)
    h = jnp.maximum(h + b1_ref[...], 0.0)
    h = jnp.maximum(
        jnp.dot(h, w2_ref[...], preferred_element_type=jnp.float32) + b2_ref[...],
        0.0)
    o_ref[...] = (
        jnp.dot(h, w3_ref[...], preferred_element_type=jnp.float32) + b3_ref[...])


def _mlp(emb, x_num, w1a, w1b, b1, w2, b2, w3, b3):
    grid = (B // _BB,)
    full = lambda shape: pl.BlockSpec(shape, lambda i: (0, 0))
    return pl.pallas_call(
        _mlp_body,
        grid=grid,
        in_specs=[
            pl.BlockSpec((_BB, F * D), lambda i: (i, 0)),
            pl.BlockSpec((_BB, NUM), lambda i: (i, 0)),
            full(w1a.shape),
            full(w1b.shape),
            full((1, H1)),
            full(w2.shape),
            full((1, H2)),
            full(w3.shape),
            full((1, 1)),
        ],
        out_specs=pl.BlockSpec((_BB, 1), lambda i: (i, 0)),
        out_shape=jax.ShapeDtypeStruct((B, 1), jnp.float32),
    )(emb, x_num, w1a, w1b, b1.reshape(1, H1), w2, b2.reshape(1, H2), w3,
      b3.reshape(1, 1))


_sc_relayout = _make_sc_relayout()


def kernel(x_num, x_cat, emb_tables, W1, b1, W2, b2, W3, b3):
    idx_t = x_cat.astype(jnp.int32).T
    table_t = jnp.transpose(emb_tables, (0, 2, 1))
    tail = emb_tables[:, _VT_FULL * 128:, :].reshape(F * _TAIL * D)
    table_lin = _sc_relayout(table_t, tail)
    emb = _sc_gather(table_lin.reshape(F, V, D), idx_t)
    w1a = W1[:F * D]
    w1b = W1[F * D:]
    return _mlp(emb, x_num, w1a, w1b, b1, W2, b2, W3, b3)


# speedup vs baseline: 1.9406x; 1.7909x over previous
import functools

import jax
import jax.numpy as jnp
from jax import lax
from jax.experimental import pallas as pl
from jax.experimental.pallas import tpu as pltpu
from jax.experimental.pallas import tpu_sc as plsc

B = 16384
F = 26
V = 100000
D = 16
NUM = 13
H1 = 128
H2 = 64
N = B * F


_VT_FULL = V // 128
_TAIL = V - _VT_FULL * 128
_G = 11
_GROUPS_PER_F = _VT_FULL // _G
_NGROUPS = F * _GROUPS_PER_F
_GW = 128 * _G
_GFLOATS = _GW * D


def _make_sc_relayout():
    info = plsc.get_sparse_core_info()
    nc, ns = info.num_cores, info.num_subcores
    nw = nc * ns
    per_w = -(-_NGROUPS // nw)
    mesh = plsc.VectorSubcoreMesh(core_axis_name="c", subcore_axis_name="s")

    @functools.partial(
        pl.kernel,
        mesh=mesh,
        out_type=jax.ShapeDtypeStruct((F * V * D,), jnp.float32),
        scratch_types=[
            pltpu.VMEM((D, _GW), jnp.float32),
            pltpu.VMEM((D, _GW), jnp.float32),
            pltpu.VMEM((_GFLOATS,), jnp.float32),
            pltpu.VMEM((_GFLOATS,), jnp.float32),
            pltpu.VMEM((_TAIL * D,), jnp.float32),
            pltpu.SemaphoreType.DMA,
            pltpu.SemaphoreType.DMA,
            pltpu.SemaphoreType.DMA,
            pltpu.SemaphoreType.DMA,
        ],
        compiler_params=pltpu.CompilerParams(use_tc_tiling_on_sc=True,
                                             needs_layout_passes=False),
    )
    def relayout_k(table_hbm, tail_hbm, out_hbm, t0, t1, r0, r1, tail_v,
                   si0, si1, so0, so1):
        wid = lax.axis_index("s") * nc + lax.axis_index("c")
        g0 = wid * per_w
        nj = jnp.minimum(per_w, _NGROUPS - g0)
        tiles = (t0, t1)
        rows = (r0, r1)
        sin = (si0, si1)
        sout = (so0, so1)
        iota = lax.iota(jnp.int32, 16)

        def src(h):
            f = h // _GROUPS_PER_F
            v0 = (h % _GROUPS_PER_F) * _GW
            return table_hbm.at[f, :, pl.ds(v0, _GW)]

        def dst(h):
            f = h // _GROUPS_PER_F
            v0 = (h % _GROUPS_PER_F) * _GW
            return out_hbm.at[pl.ds(f * (V * D) + v0 * D, _GFLOATS)]

        def wait_in(b):
            pltpu.make_async_copy(src(0), tiles[b], sin[b]).wait()

        def wait_out(b):
            pltpu.make_async_copy(rows[b], out_hbm.at[pl.ds(0, _GFLOATS)],
                                  sout[b]).wait()

        def extract(b):
            tv, rv = tiles[b], rows[b]

            def sub(t, carry):
                base = t * (128 * D)
                for d in range(D):
                    for k in range(8):
                        vec = tv[d, pl.ds(t * 128 + 16 * k, 16)]
                        idx = iota * D + (256 * k + d) + base
                        plsc.store_scatter(rv, [idx], vec)
                return carry

            lax.fori_loop(0, _G, sub, 0)

        pltpu.async_copy(src(g0), tiles[0], sin[0])

        def body(jj, carry):
            for b in range(2):
                j = 2 * jj + b
                h = g0 + j

                @pl.when(j + 1 < nj)
                def _():
                    pltpu.async_copy(src(h + 1), tiles[1 - b], sin[1 - b])

                wait_in(b)

                @pl.when(j >= 2)
                def _():
                    wait_out(b)

                extract(b)
                pltpu.async_copy(rows[b], dst(h), sout[b])
            return carry

        lax.fori_loop(0, nj // 2, body, 0)
        wait_out(0)
        wait_out(1)

        @pl.when(wid < F)
        def _():
            pltpu.sync_copy(tail_hbm.at[pl.ds(wid * (_TAIL * D), _TAIL * D)],
                            tail_v)
            pltpu.sync_copy(
                tail_v,
                out_hbm.at[pl.ds(wid * (V * D) + _VT_FULL * 128 * D,
                                 _TAIL * D)])

    return relayout_k


_CHUNK = 128


def _make_sc_gather():
    info = plsc.get_sparse_core_info()
    nc, ns = info.num_cores, info.num_subcores
    nw = nc * ns
    per_w = N // nw
    n_chunks = per_w // _CHUNK
    chunks_per_field = B // _CHUNK
    mesh = plsc.VectorSubcoreMesh(core_axis_name="c", subcore_axis_name="s")

    @functools.partial(
        pl.kernel,
        mesh=mesh,
        out_type=jax.ShapeDtypeStruct((B, F * D), jnp.float32),
        scratch_types=[
            pltpu.VMEM((_CHUNK,), jnp.int32),
            pltpu.VMEM((_CHUNK,), jnp.int32),
            pltpu.VMEM((_CHUNK, D), jnp.float32),
            pltpu.VMEM((_CHUNK, D), jnp.float32),
            pltpu.SemaphoreType.DMA,
            pltpu.SemaphoreType.DMA,
        ],
        compiler_params=pltpu.CompilerParams(use_tc_tiling_on_sc=False),
    )
    def gather_k(table_hbm, idx_hbm, out_hbm, idx0, idx1, rows0, rows1,
                 sem0, sem1):
        wid = lax.axis_index("s") * nc + lax.axis_index("c")
        chunk0 = wid * n_chunks
        idxs = (idx0, idx1)
        rows = (rows0, rows1)
        sems = (sem0, sem1)

        def stage_and_gather(c, b):
            f = c // chunks_per_field
            r0 = (c % chunks_per_field) * _CHUNK
            pltpu.sync_copy(idx_hbm.at[f, pl.ds(r0, _CHUNK)], idxs[b])
            pltpu.async_copy(table_hbm.at[f].at[idxs[b]], rows[b], sems[b])

        def finish(c, b):
            f = c // chunks_per_field
            r0 = (c % chunks_per_field) * _CHUNK
            pltpu.make_async_copy(table_hbm.at[f].at[idxs[b]], rows[b],
                                  sems[b]).wait()
            pltpu.sync_copy(rows[b],
                            out_hbm.at[pl.ds(r0, _CHUNK), pl.ds(f * D, D)])

        stage_and_gather(chunk0, 0)

        def body(jj, carry):
            for b in range(2):
                j = jj * 2 + b

                @pl.when(j + 1 < n_chunks)
                def _():
                    stage_and_gather(chunk0 + j + 1, 1 - b)

                finish(chunk0 + j, b)
            return carry

        lax.fori_loop(0, n_chunks // 2, body, 0)

    return gather_k


_sc_gather = _make_sc_gather()


_BB = 1024


def _mlp_body(emb_ref, xn_ref, w1a_ref, w1b_ref, b1_ref, w2_ref, b2_ref,
              w3_ref, b3_ref, o_ref):
    h = jnp.dot(emb_ref[...], w1a_ref[...], preferred_element_type=jnp.float32)
    h = h + jnp.dot(xn_ref[...], w1b_ref[...], preferred_element_type=jnp.float32)
    h = jnp.maximum(h + b1_ref[...], 0.0)
    h = jnp.maximum(
        jnp.dot(h, w2_ref[...], preferred_element_type=jnp.float32) + b2_ref[...],
        0.0)
    o_ref[...] = (
        jnp.dot(h, w3_ref[...], preferred_element_type=jnp.float32) + b3_ref[...])


def _mlp(emb, x_num, w1a, w1b, b1, w2, b2, w3, b3):
    grid = (B // _BB,)
    full = lambda shape: pl.BlockSpec(shape, lambda i: (0, 0))
    return pl.pallas_call(
        _mlp_body,
        grid=grid,
        in_specs=[
            pl.BlockSpec((_BB, F * D), lambda i: (i, 0)),
            pl.BlockSpec((_BB, NUM), lambda i: (i, 0)),
            full(w1a.shape),
            full(w1b.shape),
            full((1, H1)),
            full(w2.shape),
            full((1, H2)),
            full(w3.shape),
            full((1, 1)),
        ],
        out_specs=pl.BlockSpec((_BB, 1), lambda i: (i, 0)),
        out_shape=jax.ShapeDtypeStruct((B, 1), jnp.float32),
    )(emb, x_num, w1a, w1b, b1.reshape(1, H1), w2, b2.reshape(1, H2), w3,
      b3.reshape(1, 1))


_sc_relayout = _make_sc_relayout()


def kernel(x_num, x_cat, emb_tables, W1, b1, W2, b2, W3, b3):
    idx_t = x_cat.astype(jnp.int32).T
    table_t = jnp.transpose(emb_tables, (0, 2, 1))
    tail = emb_tables[:, _VT_FULL * 128:, :].reshape(F * _TAIL * D)
    table_lin = _sc_relayout(table_t, tail)
    emb = _sc_gather(table_lin.reshape(F, V, D), idx_t)
    w1a = W1[:F * D]
    w1b = W1[F * D:]
    return _mlp(emb, x_num, w1a, w1b, b1, W2, b2, W3, b3)


# speedup vs baseline: 2.3146x; 1.1927x over previous
import functools

import jax
import jax.numpy as jnp
from jax import lax
from jax.experimental import pallas as pl
from jax.experimental.pallas import tpu as pltpu
from jax.experimental.pallas import tpu_sc as plsc

B = 16384
F = 26
V = 100000
D = 16
NUM = 13
H1 = 128
H2 = 64
N = B * F


_VT_FULL = V // 128
_TAIL = V - _VT_FULL * 128
_G = 11
_GROUPS_PER_F = _VT_FULL // _G
_NGROUPS = F * _GROUPS_PER_F
_GW = 128 * _G
_GFLOATS = _GW * D


def _make_sc_relayout():
    info = plsc.get_sparse_core_info()
    nc, ns = info.num_cores, info.num_subcores
    nw = nc * ns
    per_w = -(-_NGROUPS // nw)
    mesh = plsc.VectorSubcoreMesh(core_axis_name="c", subcore_axis_name="s")

    @functools.partial(
        pl.kernel,
        mesh=mesh,
        out_type=jax.ShapeDtypeStruct((F * V * D,), jnp.float32),
        scratch_types=[
            pltpu.VMEM((D, _GW), jnp.float32),
            pltpu.VMEM((D, _GW), jnp.float32),
            pltpu.VMEM((_GFLOATS,), jnp.float32),
            pltpu.VMEM((_GFLOATS,), jnp.float32),
            pltpu.VMEM((_TAIL * D,), jnp.float32),
            pltpu.SemaphoreType.DMA,
            pltpu.SemaphoreType.DMA,
            pltpu.SemaphoreType.DMA,
            pltpu.SemaphoreType.DMA,
        ],
        compiler_params=pltpu.CompilerParams(use_tc_tiling_on_sc=True,
                                             needs_layout_passes=False),
    )
    def relayout_k(table_hbm, tail_hbm, out_hbm, t0, t1, r0, r1, tail_v,
                   si0, si1, so0, so1):
        wid = lax.axis_index("s") * nc + lax.axis_index("c")
        g0 = wid * per_w
        nj = jnp.minimum(per_w, _NGROUPS - g0)
        tiles = (t0, t1)
        rows = (r0, r1)
        sin = (si0, si1)
        sout = (so0, so1)
        iota = lax.iota(jnp.int32, 16)

        def src(h):
            f = h // _GROUPS_PER_F
            v0 = (h % _GROUPS_PER_F) * _GW
            return table_hbm.at[f, :, pl.ds(v0, _GW)]

        def dst(h):
            f = h // _GROUPS_PER_F
            v0 = (h % _GROUPS_PER_F) * _GW
            return out_hbm.at[pl.ds(f * (V * D) + v0 * D, _GFLOATS)]

        def wait_in(b):
            pltpu.make_async_copy(src(0), tiles[b], sin[b]).wait()

        def wait_out(b):
            pltpu.make_async_copy(rows[b], out_hbm.at[pl.ds(0, _GFLOATS)],
                                  sout[b]).wait()

        rot = [(iota + s) & 15 for s in range(16)]
        rot16i = [r * D + iota for r in rot]

        def extract(b):
            tv, rv = tiles[b], rows[b]

            def sub(cb, carry):
                for s in range(16):
                    col = cb * 16 + rot[s]
                    vec = plsc.load_gather(tv, [iota, col])
                    plsc.store_scatter(rv, [cb * 256 + rot16i[s]], vec)
                return carry

            lax.fori_loop(0, _GW // 16, sub, 0)

        pltpu.async_copy(src(g0), tiles[0], sin[0])

        def body(jj, carry):
            for b in range(2):
                j = 2 * jj + b
                h = g0 + j

                @pl.when(j + 1 < nj)
                def _():
                    pltpu.async_copy(src(h + 1), tiles[1 - b], sin[1 - b])

                wait_in(b)

                @pl.when(j >= 2)
                def _():
                    wait_out(b)

                extract(b)
                pltpu.async_copy(rows[b], dst(h), sout[b])
            return carry

        lax.fori_loop(0, nj // 2, body, 0)
        wait_out(0)
        wait_out(1)

        @pl.when(wid < F)
        def _():
            pltpu.sync_copy(tail_hbm.at[pl.ds(wid * (_TAIL * D), _TAIL * D)],
                            tail_v)
            pltpu.sync_copy(
                tail_v,
                out_hbm.at[pl.ds(wid * (V * D) + _VT_FULL * 128 * D,
                                 _TAIL * D)])

    return relayout_k


_CHUNK = 128


def _make_sc_gather():
    info = plsc.get_sparse_core_info()
    nc, ns = info.num_cores, info.num_subcores
    nw = nc * ns
    per_w = N // nw
    n_chunks = per_w // _CHUNK
    chunks_per_field = B // _CHUNK
    mesh = plsc.VectorSubcoreMesh(core_axis_name="c", subcore_axis_name="s")

    @functools.partial(
        pl.kernel,
        mesh=mesh,
        out_type=jax.ShapeDtypeStruct((B, F * D), jnp.float32),
        scratch_types=[
            pltpu.VMEM((_CHUNK,), jnp.int32),
            pltpu.VMEM((_CHUNK,), jnp.int32),
            pltpu.VMEM((_CHUNK, D), jnp.float32),
            pltpu.VMEM((_CHUNK, D), jnp.float32),
            pltpu.SemaphoreType.DMA,
            pltpu.SemaphoreType.DMA,
        ],
        compiler_params=pltpu.CompilerParams(use_tc_tiling_on_sc=False),
    )
    def gather_k(table_hbm, idx_hbm, out_hbm, idx0, idx1, rows0, rows1,
                 sem0, sem1):
        wid = lax.axis_index("s") * nc + lax.axis_index("c")
        chunk0 = wid * n_chunks
        idxs = (idx0, idx1)
        rows = (rows0, rows1)
        sems = (sem0, sem1)

        def stage_and_gather(c, b):
            f = c // chunks_per_field
            r0 = (c % chunks_per_field) * _CHUNK
            pltpu.sync_copy(idx_hbm.at[f, pl.ds(r0, _CHUNK)], idxs[b])
            pltpu.async_copy(table_hbm.at[f].at[idxs[b]], rows[b], sems[b])

        def finish(c, b):
            f = c // chunks_per_field
            r0 = (c % chunks_per_field) * _CHUNK
            pltpu.make_async_copy(table_hbm.at[f].at[idxs[b]], rows[b],
                                  sems[b]).wait()
            pltpu.sync_copy(rows[b],
                            out_hbm.at[pl.ds(r0, _CHUNK), pl.ds(f * D, D)])

        stage_and_gather(chunk0, 0)

        def body(jj, carry):
            for b in range(2):
                j = jj * 2 + b

                @pl.when(j + 1 < n_chunks)
                def _():
                    stage_and_gather(chunk0 + j + 1, 1 - b)

                finish(chunk0 + j, b)
            return carry

        lax.fori_loop(0, n_chunks // 2, body, 0)

    return gather_k


_sc_gather = _make_sc_gather()


_BB = 1024


def _mlp_body(emb_ref, xn_ref, w1a_ref, w1b_ref, b1_ref, w2_ref, b2_ref,
              w3_ref, b3_ref, o_ref):
    h = jnp.dot(emb_ref[...], w1a_ref[...], preferred_element_type=jnp.float32)
    h = h + jnp.dot(xn_ref[...], w1b_ref[...], preferred_element_type=jnp.float32)
    h = jnp.maximum(h + b1_ref[...], 0.0)
    h = jnp.maximum(
        jnp.dot(h, w2_ref[...], preferred_element_type=jnp.float32) + b2_ref[...],
        0.0)
    o_ref[...] = (
        jnp.dot(h, w3_ref[...], preferred_element_type=jnp.float32) + b3_ref[...])


def _mlp(emb, x_num, w1a, w1b, b1, w2, b2, w3, b3):
    grid = (B // _BB,)
    full = lambda shape: pl.BlockSpec(shape, lambda i: (0, 0))
    return pl.pallas_call(
        _mlp_body,
        grid=grid,
        in_specs=[
            pl.BlockSpec((_BB, F * D), lambda i: (i, 0)),
            pl.BlockSpec((_BB, NUM), lambda i: (i, 0)),
            full(w1a.shape),
            full(w1b.shape),
            full((1, H1)),
            full(w2.shape),
            full((1, H2)),
            full(w3.shape),
            full((1, 1)),
        ],
        out_specs=pl.BlockSpec((_BB, 1), lambda i: (i, 0)),
        out_shape=jax.ShapeDtypeStruct((B, 1), jnp.float32),
    )(emb, x_num, w1a, w1b, b1.reshape(1, H1), w2, b2.reshape(1, H2), w3,
      b3.reshape(1, 1))


_sc_relayout = _make_sc_relayout()


def kernel(x_num, x_cat, emb_tables, W1, b1, W2, b2, W3, b3):
    idx_t = x_cat.astype(jnp.int32).T
    table_t = jnp.transpose(emb_tables, (0, 2, 1))
    tail = emb_tables[:, _VT_FULL * 128:, :].reshape(F * _TAIL * D)
    table_lin = _sc_relayout(table_t, tail)
    emb = _sc_gather(table_lin.reshape(F, V, D), idx_t)
    w1a = W1[:F * D]
    w1b = W1[F * D:]
    return _mlp(emb, x_num, w1a, w1b, b1, W2, b2, W3, b3)


# speedup vs baseline: 2.5907x; 1.1193x over previous
import functools

import jax
import jax.numpy as jnp
from jax import lax
from jax.experimental import pallas as pl
from jax.experimental.pallas import tpu as pltpu
from jax.experimental.pallas import tpu_sc as plsc

B = 16384
F = 26
V = 100000
D = 16
NUM = 13
H1 = 128
H2 = 64
N = B * F


_VT_FULL = V // 128
_TAIL = V - _VT_FULL * 128
_G = 11
_GROUPS_PER_F = _VT_FULL // _G
_NGROUPS = F * _GROUPS_PER_F
_GW = 128 * _G
_GFLOATS = _GW * D


def _make_sc_relayout():
    info = plsc.get_sparse_core_info()
    nc, ns = info.num_cores, info.num_subcores
    nw = nc * ns
    per_w = -(-_NGROUPS // nw)
    mesh = plsc.VectorSubcoreMesh(core_axis_name="c", subcore_axis_name="s")

    @functools.partial(
        pl.kernel,
        mesh=mesh,
        out_type=jax.ShapeDtypeStruct((F * V * D,), jnp.float32),
        scratch_types=[
            pltpu.VMEM((D, _GW), jnp.float32),
            pltpu.VMEM((D, _GW), jnp.float32),
            pltpu.VMEM((_GFLOATS,), jnp.float32),
            pltpu.VMEM((_GFLOATS,), jnp.float32),
            pltpu.VMEM((_TAIL * D,), jnp.float32),
            pltpu.SemaphoreType.DMA,
            pltpu.SemaphoreType.DMA,
            pltpu.SemaphoreType.DMA,
            pltpu.SemaphoreType.DMA,
        ],
        compiler_params=pltpu.CompilerParams(use_tc_tiling_on_sc=True,
                                             needs_layout_passes=False),
    )
    def relayout_k(table_hbm, tail_hbm, out_hbm, t0, t1, r0, r1, tail_v,
                   si0, si1, so0, so1):
        wid = lax.axis_index("s") * nc + lax.axis_index("c")
        g0 = wid * per_w
        nj = jnp.minimum(per_w, _NGROUPS - g0)
        tiles = (t0, t1)
        rows = (r0, r1)
        sin = (si0, si1)
        sout = (so0, so1)
        iota = lax.iota(jnp.int32, 16)

        def src(h):
            f = h // _GROUPS_PER_F
            v0 = (h % _GROUPS_PER_F) * _GW
            return table_hbm.at[f, :, pl.ds(v0, _GW)]

        def dst(h):
            f = h // _GROUPS_PER_F
            v0 = (h % _GROUPS_PER_F) * _GW
            return out_hbm.at[pl.ds(f * (V * D) + v0 * D, _GFLOATS)]

        def wait_in(b):
            pltpu.make_async_copy(src(0), tiles[b], sin[b]).wait()

        def wait_out(b):
            pltpu.make_async_copy(rows[b], out_hbm.at[pl.ds(0, _GFLOATS)],
                                  sout[b]).wait()

        rot = [(iota + s) & 15 for s in range(16)]
        rot16i = [r * D + iota for r in rot]

        def extract(b):
            tv, rv = tiles[b], rows[b]

            def sub(cb2, carry):
                for u in range(2):
                    cb = cb2 * 2 + u
                    for s in range(16):
                        col = cb * 16 + rot[s]
                        vec = plsc.load_gather(tv, [iota, col])
                        plsc.store_scatter(rv, [cb * 256 + rot16i[s]], vec)
                return carry

            lax.fori_loop(0, _GW // 32, sub, 0)

        pltpu.async_copy(src(g0), tiles[0], sin[0])

        def body(jj, carry):
            for b in range(2):
                j = 2 * jj + b
                h = g0 + j

                @pl.when(j + 1 < nj)
                def _():
                    pltpu.async_copy(src(h + 1), tiles[1 - b], sin[1 - b])

                wait_in(b)

                @pl.when(j >= 2)
                def _():
                    wait_out(b)

                extract(b)
                pltpu.async_copy(rows[b], dst(h), sout[b])
            return carry

        lax.fori_loop(0, nj // 2, body, 0)
        wait_out(0)
        wait_out(1)

        @pl.when(wid < F)
        def _():
            pltpu.sync_copy(tail_hbm.at[pl.ds(wid * (_TAIL * D), _TAIL * D)],
                            tail_v)
            pltpu.sync_copy(
                tail_v,
                out_hbm.at[pl.ds(wid * (V * D) + _VT_FULL * 128 * D,
                                 _TAIL * D)])

    return relayout_k


_CHUNK = 128


def _make_sc_gather():
    info = plsc.get_sparse_core_info()
    nc, ns = info.num_cores, info.num_subcores
    nw = nc * ns
    per_w = N // nw
    n_chunks = per_w // _CHUNK
    chunks_per_field = B // _CHUNK
    mesh = plsc.VectorSubcoreMesh(core_axis_name="c", subcore_axis_name="s")

    @functools.partial(
        pl.kernel,
        mesh=mesh,
        out_type=jax.ShapeDtypeStruct((B, F * D), jnp.float32),
        scratch_types=[
            pltpu.VMEM((N // nw,), jnp.int32),
            pltpu.VMEM((_CHUNK, D), jnp.float32),
            pltpu.VMEM((_CHUNK, D), jnp.float32),
            pltpu.VMEM((_CHUNK, D), jnp.float32),
            pltpu.VMEM((_CHUNK, D), jnp.float32),
            pltpu.SemaphoreType.DMA,
            pltpu.SemaphoreType.DMA,
            pltpu.SemaphoreType.DMA,
            pltpu.SemaphoreType.DMA,
            pltpu.SemaphoreType.DMA,
            pltpu.SemaphoreType.DMA,
            pltpu.SemaphoreType.DMA,
            pltpu.SemaphoreType.DMA,
        ],
        compiler_params=pltpu.CompilerParams(use_tc_tiling_on_sc=False),
    )
    def gather_k(table_hbm, idx_hbm, out_hbm, idx_all, r0, r1, r2, r3,
                 sg0, sg1, sg2, sg3, so0, so1, so2, so3):
        wid = lax.axis_index("s") * nc + lax.axis_index("c")
        chunk0 = wid * n_chunks
        base_w = wid * per_w
        rows = (r0, r1, r2, r3)
        sg = (sg0, sg1, sg2, sg3)
        so = (so0, so1, so2, so3)
        pltpu.sync_copy(idx_hbm.at[pl.ds(base_w, per_w)], idx_all)

        def gstart(j, b):
            c = chunk0 + j
            f = c // chunks_per_field
            pltpu.async_copy(
                table_hbm.at[f].at[idx_all.at[pl.ds(j * _CHUNK, _CHUNK)]],
                rows[b], sg[b])

        def gwait(b):
            pltpu.make_async_copy(
                table_hbm.at[0].at[idx_all.at[pl.ds(0, _CHUNK)]], rows[b],
                sg[b]).wait()

        def ostart(j, b):
            c = chunk0 + j
            f = c // chunks_per_field
            rr = (c % chunks_per_field) * _CHUNK
            pltpu.async_copy(rows[b],
                             out_hbm.at[pl.ds(rr, _CHUNK), pl.ds(f * D, D)],
                             so[b])

        def owait(b):
            pltpu.make_async_copy(rows[b],
                                  out_hbm.at[pl.ds(0, _CHUNK), pl.ds(0, D)],
                                  so[b]).wait()

        for b in range(3):
            gstart(b, b)

        def body(jj, carry):
            for b in range(4):
                j = 4 * jj + b
                bn = (b + 3) % 4
                gwait(b)
                ostart(j, b)

                @pl.when(j + 3 < n_chunks)
                def _():
                    @pl.when(j >= 1)
                    def _():
                        owait(bn)

                    gstart(j + 3, bn)
            return carry

        lax.fori_loop(0, n_chunks // 4, body, 0)
        for b in range(4):
            owait(b)

    return gather_k


_sc_gather = _make_sc_gather()


_BB = 1024


def _mlp_body(emb_ref, xn_ref, w1a_ref, w1b_ref, b1_ref, w2_ref, b2_ref,
              w3_ref, b3_ref, o_ref):
    h = jnp.dot(emb_ref[...], w1a_ref[...], preferred_element_type=jnp.float32)
    h = h + jnp.dot(xn_ref[...], w1b_ref[...], preferred_element_type=jnp.float32)
    h = jnp.maximum(h + b1_ref[...], 0.0)
    h = jnp.maximum(
        jnp.dot(h, w2_ref[...], preferred_element_type=jnp.float32) + b2_ref[...],
        0.0)
    o_ref[...] = (
        jnp.dot(h, w3_ref[...], preferred_element_type=jnp.float32) + b3_ref[...])


def _mlp(emb, x_num, w1a, w1b, b1, w2, b2, w3, b3):
    grid = (B // _BB,)
    full = lambda shape: pl.BlockSpec(shape, lambda i: (0, 0))
    return pl.pallas_call(
        _mlp_body,
        grid=grid,
        in_specs=[
            pl.BlockSpec((_BB, F * D), lambda i: (i, 0)),
            pl.BlockSpec((_BB, NUM), lambda i: (i, 0)),
            full(w1a.shape),
            full(w1b.shape),
            full((1, H1)),
            full(w2.shape),
            full((1, H2)),
            full(w3.shape),
            full((1, 1)),
        ],
        out_specs=pl.BlockSpec((_BB, 1), lambda i: (i, 0)),
        out_shape=jax.ShapeDtypeStruct((B, 1), jnp.float32),
    )(emb, x_num, w1a, w1b, b1.reshape(1, H1), w2, b2.reshape(1, H2), w3,
      b3.reshape(1, 1))


_sc_relayout = _make_sc_relayout()


def kernel(x_num, x_cat, emb_tables, W1, b1, W2, b2, W3, b3):
    idx_t = x_cat.astype(jnp.int32).T
    table_t = jnp.transpose(emb_tables, (0, 2, 1))
    tail = emb_tables[:, _VT_FULL * 128:, :].reshape(F * _TAIL * D)
    table_lin = _sc_relayout(table_t, tail)
    emb = _sc_gather(table_lin.reshape(F, V, D), idx_t.reshape(N))
    w1a = W1[:F * D]
    w1b = W1[F * D:]
    return _mlp(emb, x_num, w1a, w1b, b1, W2, b2, W3, b3)


# speedup vs baseline: 2.5920x; 1.0005x over previous
import functools

import jax
import jax.numpy as jnp
from jax import lax
from jax.experimental import pallas as pl
from jax.experimental.pallas import tpu as pltpu
from jax.experimental.pallas import tpu_sc as plsc

B = 16384
F = 26
V = 100000
D = 16
NUM = 13
H1 = 128
H2 = 64
N = B * F


_VT_FULL = V // 128
_TAIL = V - _VT_FULL * 128
_G = 11
_GROUPS_PER_F = _VT_FULL // _G
_NGROUPS = F * _GROUPS_PER_F
_GW = 128 * _G
_GFLOATS = _GW * D


def _make_sc_relayout():
    info = plsc.get_sparse_core_info()
    nc, ns = info.num_cores, info.num_subcores
    nw = nc * ns
    per_w = -(-_NGROUPS // nw)
    mesh = plsc.VectorSubcoreMesh(core_axis_name="c", subcore_axis_name="s")

    @functools.partial(
        pl.kernel,
        mesh=mesh,
        out_type=jax.ShapeDtypeStruct((F * V * D,), jnp.float32),
        scratch_types=[
            pltpu.VMEM((D, _GW), jnp.float32),
            pltpu.VMEM((D, _GW), jnp.float32),
            pltpu.VMEM((_GFLOATS,), jnp.float32),
            pltpu.VMEM((_GFLOATS,), jnp.float32),
            pltpu.VMEM((_TAIL * D,), jnp.float32),
            pltpu.SemaphoreType.DMA,
            pltpu.SemaphoreType.DMA,
            pltpu.SemaphoreType.DMA,
            pltpu.SemaphoreType.DMA,
        ],
        compiler_params=pltpu.CompilerParams(use_tc_tiling_on_sc=True,
                                             needs_layout_passes=False),
    )
    def relayout_k(table_hbm, tail_hbm, out_hbm, t0, t1, r0, r1, tail_v,
                   si0, si1, so0, so1):
        wid = lax.axis_index("s") * nc + lax.axis_index("c")
        g0 = wid * per_w
        nj = jnp.minimum(per_w, _NGROUPS - g0)
        tiles = (t0, t1)
        rows = (r0, r1)
        sin = (si0, si1)
        sout = (so0, so1)
        iota = lax.iota(jnp.int32, 16)

        def src(h):
            f = h // _GROUPS_PER_F
            v0 = (h % _GROUPS_PER_F) * _GW
            return table_hbm.at[f, :, pl.ds(v0, _GW)]

        def dst(h):
            f = h // _GROUPS_PER_F
            v0 = (h % _GROUPS_PER_F) * _GW
            return out_hbm.at[pl.ds(f * (V * D) + v0 * D, _GFLOATS)]

        def wait_in(b):
            pltpu.make_async_copy(src(0), tiles[b], sin[b]).wait()

        _HF = _GFLOATS // 2

        def wait_out(b):
            pltpu.make_async_copy(rows[b].at[pl.ds(0, _HF)],
                                  out_hbm.at[pl.ds(0, _HF)], sout[b]).wait()
            pltpu.make_async_copy(rows[b].at[pl.ds(0, _HF)],
                                  out_hbm.at[pl.ds(0, _HF)], sout[b]).wait()

        rot = [(iota + s) & 15 for s in range(16)]
        rot16i = [r * D + iota for r in rot]

        def extract_half(b, half):
            tv, rv = tiles[b], rows[b]

            def sub(cb2, carry):
                for u in range(2):
                    cb = cb2 * 2 + u
                    for s in range(16):
                        col = cb * 16 + rot[s]
                        vec = plsc.load_gather(tv, [iota, col])
                        plsc.store_scatter(rv, [cb * 256 + rot16i[s]], vec)
                return carry

            n2 = _GW // 32
            lax.fori_loop(half * (n2 // 2), (half + 1) * (n2 // 2), sub, 0)

        pltpu.async_copy(src(g0), tiles[0], sin[0])

        def body(jj, carry):
            for b in range(2):
                j = 2 * jj + b
                h = g0 + j

                @pl.when(j + 1 < nj)
                def _():
                    pltpu.async_copy(src(h + 1), tiles[1 - b], sin[1 - b])

                wait_in(b)

                @pl.when(j >= 2)
                def _():
                    wait_out(b)

                extract_half(b, 0)
                pltpu.async_copy(rows[b].at[pl.ds(0, _HF)],
                                 dst(h).at[pl.ds(0, _HF)], sout[b])
                extract_half(b, 1)
                pltpu.async_copy(rows[b].at[pl.ds(_HF, _HF)],
                                 dst(h).at[pl.ds(_HF, _HF)], sout[b])
            return carry

        lax.fori_loop(0, nj // 2, body, 0)
        wait_out(0)
        wait_out(1)

        @pl.when(wid < F)
        def _():
            pltpu.sync_copy(tail_hbm.at[pl.ds(wid * (_TAIL * D), _TAIL * D)],
                            tail_v)
            pltpu.sync_copy(
                tail_v,
                out_hbm.at[pl.ds(wid * (V * D) + _VT_FULL * 128 * D,
                                 _TAIL * D)])

    return relayout_k


_CHUNK = 128


def _make_sc_gather():
    info = plsc.get_sparse_core_info()
    nc, ns = info.num_cores, info.num_subcores
    nw = nc * ns
    per_w = N // nw
    n_chunks = per_w // _CHUNK
    chunks_per_field = B // _CHUNK
    mesh = plsc.VectorSubcoreMesh(core_axis_name="c", subcore_axis_name="s")

    @functools.partial(
        pl.kernel,
        mesh=mesh,
        out_type=jax.ShapeDtypeStruct((B, F * D), jnp.float32),
        scratch_types=[
            pltpu.VMEM((N // nw,), jnp.int32),
            pltpu.VMEM((_CHUNK, D), jnp.float32),
            pltpu.VMEM((_CHUNK, D), jnp.float32),
            pltpu.VMEM((_CHUNK, D), jnp.float32),
            pltpu.VMEM((_CHUNK, D), jnp.float32),
            pltpu.SemaphoreType.DMA,
            pltpu.SemaphoreType.DMA,
            pltpu.SemaphoreType.DMA,
            pltpu.SemaphoreType.DMA,
            pltpu.SemaphoreType.DMA,
            pltpu.SemaphoreType.DMA,
            pltpu.SemaphoreType.DMA,
            pltpu.SemaphoreType.DMA,
        ],
        compiler_params=pltpu.CompilerParams(use_tc_tiling_on_sc=False),
    )
    def gather_k(table_hbm, idx_hbm, out_hbm, idx_all, r0, r1, r2, r3,
                 sg0, sg1, sg2, sg3, so0, so1, so2, so3):
        wid = lax.axis_index("s") * nc + lax.axis_index("c")
        chunk0 = wid * n_chunks
        base_w = wid * per_w
        rows = (r0, r1, r2, r3)
        sg = (sg0, sg1, sg2, sg3)
        so = (so0, so1, so2, so3)
        pltpu.sync_copy(idx_hbm.at[pl.ds(base_w, per_w)], idx_all)

        def gstart(j, b):
            c = chunk0 + j
            f = c // chunks_per_field
            pltpu.async_copy(
                table_hbm.at[f].at[idx_all.at[pl.ds(j * _CHUNK, _CHUNK)]],
                rows[b], sg[b])

        def gwait(b):
            pltpu.make_async_copy(
                table_hbm.at[0].at[idx_all.at[pl.ds(0, _CHUNK)]], rows[b],
                sg[b]).wait()

        def ostart(j, b):
            c = chunk0 + j
            f = c // chunks_per_field
            rr = (c % chunks_per_field) * _CHUNK
            pltpu.async_copy(rows[b],
                             out_hbm.at[pl.ds(rr, _CHUNK), pl.ds(f * D, D)],
                             so[b])

        def owait(b):
            pltpu.make_async_copy(rows[b],
                                  out_hbm.at[pl.ds(0, _CHUNK), pl.ds(0, D)],
                                  so[b]).wait()

        for b in range(3):
            gstart(b, b)

        def body(jj, carry):
            for b in range(4):
                j = 4 * jj + b
                bn = (b + 3) % 4
                gwait(b)
                ostart(j, b)

                @pl.when(j + 3 < n_chunks)
                def _():
                    @pl.when(j >= 1)
                    def _():
                        owait(bn)

                    gstart(j + 3, bn)
            return carry

        lax.fori_loop(0, n_chunks // 4, body, 0)
        for b in range(4):
            owait(b)

    return gather_k


_sc_gather = _make_sc_gather()


_BB = 1024


def _mlp_body(emb_ref, xn_ref, w1a_ref, w1b_ref, b1_ref, w2_ref, b2_ref,
              w3_ref, b3_ref, o_ref):
    h = jnp.dot(emb_ref[...], w1a_ref[...], preferred_element_type=jnp.float32)
    h = h + jnp.dot(xn_ref[...], w1b_ref[...], preferred_element_type=jnp.float32)
    h = jnp.maximum(h + b1_ref[...], 0.0)
    h = jnp.maximum(
        jnp.dot(h, w2_ref[...], preferred_element_type=jnp.float32) + b2_ref[...],
        0.0)
    o_ref[...] = (
        jnp.dot(h, w3_ref[...], preferred_element_type=jnp.float32) + b3_ref[...])


def _mlp(emb, x_num, w1a, w1b, b1, w2, b2, w3, b3):
    grid = (B // _BB,)
    full = lambda shape: pl.BlockSpec(shape, lambda i: (0, 0))
    return pl.pallas_call(
        _mlp_body,
        grid=grid,
        in_specs=[
            pl.BlockSpec((_BB, F * D), lambda i: (i, 0)),
            pl.BlockSpec((_BB, NUM), lambda i: (i, 0)),
            full(w1a.shape),
            full(w1b.shape),
            full((1, H1)),
            full(w2.shape),
            full((1, H2)),
            full(w3.shape),
            full((1, 1)),
        ],
        out_specs=pl.BlockSpec((_BB, 1), lambda i: (i, 0)),
        out_shape=jax.ShapeDtypeStruct((B, 1), jnp.float32),
    )(emb, x_num, w1a, w1b, b1.reshape(1, H1), w2, b2.reshape(1, H2), w3,
      b3.reshape(1, 1))


_sc_relayout = _make_sc_relayout()


def kernel(x_num, x_cat, emb_tables, W1, b1, W2, b2, W3, b3):
    idx_t = x_cat.astype(jnp.int32).T
    table_t = jnp.transpose(emb_tables, (0, 2, 1))
    tail = emb_tables[:, _VT_FULL * 128:, :].reshape(F * _TAIL * D)
    table_lin = _sc_relayout(table_t, tail)
    emb = _sc_gather(table_lin.reshape(F, V, D), idx_t.reshape(N))
    w1a = W1[:F * D]
    w1b = W1[F * D:]
    return _mlp(emb, x_num, w1a, w1b, b1, W2, b2, W3, b3)
